# baseline (speedup 1.0000x reference)
import functools

import jax
import jax.numpy as jnp
from jax import lax
from jax.experimental import pallas as pl
from jax.experimental.pallas import tpu as pltpu
from jax.experimental.pallas import tpu_sc as plsc

NC = 2
NS = 16
NW = NC * NS
LANES = 16

SENTINEL = 0x40000000

_SC_PARAMS = pltpu.CompilerParams(needs_layout_passes=False)


GCH = 64


def _make_gather(c, e):
    n_ch = e // GCH
    assert n_ch >= 12
    mesh = plsc.VectorSubcoreMesh(core_axis_name="c", subcore_axis_name="s")

    n_main = (n_ch - 9) // 6
    first_rem = 3 + 6 * n_main

    @functools.partial(
        pl.kernel,
        mesh=mesh,
        compiler_params=_SC_PARAMS,
        out_type=jax.ShapeDtypeStruct((NW * e, c), jnp.float32),
        scratch_types=[
            pltpu.VMEM((e,), jnp.int32),
        ]
        + [pltpu.VMEM((GCH, c), jnp.float32)] * 6
        + [pltpu.SemaphoreType.DMA] * 6,
    )
    def gather_k(feat_hbm, idx_hbm, out_hbm, idx_v,
                 b0, b1, b2, b3, b4, b5, s0, s1, s2, s3, s4, s5):
        bufs = (b0, b1, b2, b3, b4, b5)
        sems = (s0, s1, s2, s3, s4, s5)
        wid = lax.axis_index("s") * NC + lax.axis_index("c")
        base = wid * e
        pltpu.sync_copy(idx_hbm.at[pl.ds(base, e)], idx_v)

        def issue(ch, b):
            pltpu.async_copy(
                feat_hbm.at[idx_v.at[pl.ds(ch * GCH, GCH)]], bufs[b], sems[b]
            )

        def wait(b):
            pltpu.make_async_copy(
                feat_hbm.at[pl.ds(0, GCH)], bufs[b], sems[b]
            ).wait()

        def write(j, b):
            pltpu.async_copy(
                bufs[b], out_hbm.at[pl.ds(base + j * GCH, GCH)], sems[b]
            )

        for b in range(3):
            issue(b, b)
        for j in range(3):
            wait(j)
            write(j, j)
            issue(j + 3, (j + 3) % 6)

        def step(j, jmod):
            b = jmod % 6
            wait(b)
            write(j, b)
            b2 = (jmod + 3) % 6
            wait(b2)
            issue(j + 3, b2)

        @pl.loop(0, n_main)
        def _(t):
            for i in range(6):
                step(t * 6 + 3 + i, 3 + i)

        for j in range(first_rem, n_ch - 3):
            step(j, j % 6)
        for j in range(n_ch - 3, n_ch):
            b = j % 6
            wait(b)
            write(j, b)
        for b in range(6):
            wait(b)

    return gather_k


def _gemm(gathered, weight, k, mp, c):
    bm = 2048
    nj = mp // bm

    def body(g_ref, w_ref, o_ref):
        o_ref[...] = jnp.dot(
            g_ref[...], w_ref[0], preferred_element_type=jnp.float32
        )

    return pl.pallas_call(
        body,
        grid=(k, nj),
        in_specs=[
            pl.BlockSpec((bm, c), lambda ki, ji: (ki * nj + ji, 0)),
            pl.BlockSpec((1, c, c), lambda ki, ji: (ki, 0, 0)),
        ],
        out_specs=pl.BlockSpec((bm, c), lambda ki, ji: (ki * nj + ji, 0)),
        out_shape=jax.ShapeDtypeStruct((k * mp, c), jnp.float32),
    )(gathered, weight)


CH = 64
OMCH = 2160
BIN = 8192


def _make_scatter(c, flat, rng, n_ranges, n_pad):
    e2 = flat // NS
    n_ob = e2 // OMCH
    assert n_ob % 2 == 0
    n_grp = OMCH // LANES
    rows_per_tile = rng // NS
    n_init = rows_per_tile // CH
    bin_cap = BIN + OMCH + CH
    n_pass = (n_ranges + NC - 1) // NC
    mesh = plsc.VectorSubcoreMesh(core_axis_name="c", subcore_axis_name="s")

    @functools.partial(
        pl.kernel,
        mesh=mesh,
        compiler_params=_SC_PARAMS,
        out_type=jax.ShapeDtypeStruct((n_pad, c), jnp.float32),
        scratch_types=[
            pltpu.VMEM((OMCH,), jnp.int32),
            pltpu.VMEM((OMCH,), jnp.int32),
            pltpu.VMEM((bin_cap,), jnp.int32),
            pltpu.VMEM((bin_cap,), jnp.int32),
            pltpu.VMEM((CH,), jnp.int32),
            pltpu.VMEM((CH,), jnp.int32),
            pltpu.VMEM((CH, c), jnp.float32),
            pltpu.VMEM((CH, c), jnp.float32),
            pltpu.VMEM((c,), jnp.float32),
            pltpu.VMEM_SHARED((rng + 8, c), jnp.float32),
            pltpu.SemaphoreType.DMA,
            pltpu.SemaphoreType.DMA,
            pltpu.SemaphoreType.DMA,
            pltpu.SemaphoreType.DMA,
        ],
    )
    def scatter_k(contrib_hbm, omap_hbm, bias_hbm, out_hbm,
                  om_c0, om_c1, bin_idx, bin_off, off0, off1,
                  rows0, rows1, bias_v, spmem,
                  osem0, osem1, gsem0, gsem1):
        cid = lax.axis_index("c")
        sid = lax.axis_index("s")
        share = sid * e2
        trash = jnp.int32(rng)
        iota = lax.iota(jnp.int32, LANES)
        dump = jnp.int32(bin_cap - LANES) + iota
        om_bufs = (om_c0, om_c1)
        om_sems = (osem0, osem1)
        rows = (rows0, rows1)
        offs = (off0, off1)
        gsems = (gsem0, gsem1)

        pltpu.sync_copy(bias_hbm, bias_v)

        def g_issue(jc, b):
            pltpu.async_copy(
                contrib_hbm.at[bin_idx.at[pl.ds(jc * CH, CH)]],
                rows[b], gsems[b],
            )

        def g_wait(b):
            pltpu.make_async_copy(
                contrib_hbm.at[pl.ds(0, CH)], rows[b], gsems[b]
            ).wait()

        def do_add(jc, b):
            for i in range(CH // LANES):
                offs[b][pl.ds(i * LANES, LANES)] = bin_off[
                    pl.ds(jc * CH + i * LANES, LANES)
                ]
            pltpu.sync_copy(rows[b], spmem.at[offs[b]], add=True)

        def emit(nch):
            @pl.when(nch > 0)
            def _():
                g_issue(0, 0)

            @pl.loop(0, (nch + 1) // 2)
            def _(t):
                j0 = 2 * t
                j1 = j0 + 1

                @pl.when(j1 < nch)
                def _():
                    g_issue(j1, 1)

                g_wait(0)
                do_add(j0, 0)

                @pl.when(j1 < nch)
                def _():
                    @pl.when(j1 + 1 < nch)
                    def _():
                        g_issue(j1 + 1, 0)

                    g_wait(1)
                    do_add(j1, 1)

        def pad_and_emit(cnt):
            for i in range(CH // LANES):
                pos = cnt + i * LANES + iota
                plsc.store_scatter(bin_idx, [pos],
                                   jnp.zeros((LANES,), jnp.int32))
                plsc.store_scatter(bin_off, [pos],
                                   jnp.full((LANES,), trash, jnp.int32))
            emit((cnt + CH - 1) // CH)

        def om_issue(ob, h):
            pltpu.async_copy(
                omap_hbm.at[pl.ds(share + ob * OMCH, OMCH)],
                om_bufs[h], om_sems[h],
            )

        def om_wait(h):
            pltpu.make_async_copy(
                omap_hbm.at[pl.ds(0, OMCH)], om_bufs[h], om_sems[h]
            ).wait()

        @pl.loop(0, n_pass)
        def _(p):
            r = p * NC + cid

            @pl.when(r < n_ranges)
            def _():
                base = r * rng

                @pl.loop(0, CH)
                def _(i):
                    for j in range(c // LANES):
                        rows0[i, pl.ds(j * LANES, LANES)] = bias_v[
                            pl.ds(j * LANES, LANES)
                        ]

                for ch in range(n_init):
                    pltpu.sync_copy(
                        rows0,
                        spmem.at[pl.ds(sid * rows_per_tile + ch * CH, CH)],
                    )
                plsc.subcore_barrier()

                om_issue(0, 0)

                @pl.loop(0, n_ob // 2,
                         init_carry=jnp.zeros((LANES,), jnp.int32))
                def scan(u, cnt_v):
                    for h in range(2):
                        ob = 2 * u + h

                        @pl.when(ob + 1 < n_ob)
                        def _():
                            om_issue(ob + 1, 1 - h)

                        om_wait(h)
                        om_c = om_bufs[h]

                        @pl.loop(0, n_grp, init_carry=cnt_v)
                        def grp(g, cnt_v):
                            v = om_c[pl.ds(g * LANES, LANES)]
                            local = v - base
                            mask = (local >= 0) & (local < rng)
                            gidx = share + ob * OMCH + g * LANES + iota
                            csum = plsc.cumsum(mask.astype(jnp.int32))
                            pos = jnp.where(mask, cnt_v + csum - 1, dump)
                            plsc.store_scatter(bin_idx, [pos], gidx)
                            plsc.store_scatter(
                                bin_off, [pos],
                                jnp.where(mask, local, trash),
                            )
                            return cnt_v + plsc.all_reduce_population_count(
                                mask
                            )

                        cnt_v = grp
                        s = jnp.max(cnt_v)

                        @pl.when(s >= BIN)
                        def _():
                            pad_and_emit(s)

                        cnt_v = jnp.where(
                            jnp.broadcast_to(s >= BIN, (LANES,)),
                            jnp.zeros((LANES,), jnp.int32),
                            cnt_v,
                        )
                    return cnt_v

                pad_and_emit(jnp.max(scan))
                plsc.subcore_barrier()

                row0 = sid * rows_per_tile
                pltpu.sync_copy(
                    spmem.at[pl.ds(row0, rows_per_tile)],
                    out_hbm.at[pl.ds(base + row0, rows_per_tile)],
                )

    return scatter_k


def kernel(features, in_map, out_map, weight, bias):
    n_feat, c = features.shape
    k, m = in_map.shape

    mp = m
    while (k * mp) % (128 * NW) or (k * mp // NS) % OMCH:
        mp += 1
    flat = k * mp
    e = flat // NW

    rng = 10240
    n_ranges = -(-n_feat // rng)
    n_pad = n_ranges * rng

    in_p = jnp.pad(in_map, ((0, 0), (0, mp - m))).reshape(-1)
    om_p = jnp.pad(
        out_map, ((0, 0), (0, mp - m)), constant_values=SENTINEL
    ).reshape(-1)

    gathered = _make_gather(c, e)(features, in_p)
    contrib = _gemm(gathered, weight, k, mp, c)
    out_pad = _make_scatter(c, flat, rng, n_ranges, n_pad)(
        contrib, om_p, bias
    )
    return out_pad[:n_feat]


# speedup vs baseline: 1.0285x; 1.0285x over previous
import functools

import jax
import jax.numpy as jnp
from jax import lax
from jax.experimental import pallas as pl
from jax.experimental.pallas import tpu as pltpu
from jax.experimental.pallas import tpu_sc as plsc

NC = 2
NS = 16
NW = NC * NS
LANES = 16

SENTINEL = 0x40000000

_SC_PARAMS = pltpu.CompilerParams(needs_layout_passes=False)


GCH = 128


def _make_gather(c, e):
    n_ch = e // GCH
    assert n_ch >= 12
    mesh = plsc.VectorSubcoreMesh(core_axis_name="c", subcore_axis_name="s")

    n_main = (n_ch - 9) // 6
    first_rem = 3 + 6 * n_main

    @functools.partial(
        pl.kernel,
        mesh=mesh,
        compiler_params=_SC_PARAMS,
        out_type=jax.ShapeDtypeStruct((NW * e, c), jnp.float32),
        scratch_types=[
            pltpu.VMEM((e,), jnp.int32),
        ]
        + [pltpu.VMEM((GCH, c), jnp.float32)] * 6
        + [pltpu.SemaphoreType.DMA] * 6,
    )
    def gather_k(feat_hbm, idx_hbm, out_hbm, idx_v,
                 b0, b1, b2, b3, b4, b5, s0, s1, s2, s3, s4, s5):
        bufs = (b0, b1, b2, b3, b4, b5)
        sems = (s0, s1, s2, s3, s4, s5)
        wid = lax.axis_index("s") * NC + lax.axis_index("c")
        base = wid * e
        pltpu.sync_copy(idx_hbm.at[pl.ds(base, e)], idx_v)

        def issue(ch, b):
            pltpu.async_copy(
                feat_hbm.at[idx_v.at[pl.ds(ch * GCH, GCH)]], bufs[b], sems[b]
            )

        def wait(b):
            pltpu.make_async_copy(
                feat_hbm.at[pl.ds(0, GCH)], bufs[b], sems[b]
            ).wait()

        def write(j, b):
            pltpu.async_copy(
                bufs[b], out_hbm.at[pl.ds(base + j * GCH, GCH)], sems[b]
            )

        for b in range(3):
            issue(b, b)
        for j in range(3):
            wait(j)
            write(j, j)
            issue(j + 3, (j + 3) % 6)

        def step(j, jmod):
            b = jmod % 6
            wait(b)
            write(j, b)
            b2 = (jmod + 3) % 6
            wait(b2)
            issue(j + 3, b2)

        @pl.loop(0, n_main)
        def _(t):
            for i in range(6):
                step(t * 6 + 3 + i, 3 + i)

        for j in range(first_rem, n_ch - 3):
            step(j, j % 6)
        for j in range(n_ch - 3, n_ch):
            b = j % 6
            wait(b)
            write(j, b)
        for b in range(6):
            wait(b)

    return gather_k


def _gemm(gathered, weight, k, mp, c):
    bm = 2048
    nj = mp // bm

    def body(g_ref, w_ref, o_ref):
        o_ref[...] = jnp.dot(
            g_ref[...], w_ref[0], preferred_element_type=jnp.float32
        )

    return pl.pallas_call(
        body,
        grid=(k, nj),
        in_specs=[
            pl.BlockSpec((bm, c), lambda ki, ji: (ki * nj + ji, 0)),
            pl.BlockSpec((1, c, c), lambda ki, ji: (ki, 0, 0)),
        ],
        out_specs=pl.BlockSpec((bm, c), lambda ki, ji: (ki * nj + ji, 0)),
        out_shape=jax.ShapeDtypeStruct((k * mp, c), jnp.float32),
    )(gathered, weight)


CH = 64
OMCH = 2160
BIN = 7168


def _make_scatter(c, flat, rng, n_ranges, n_pad):
    e2 = flat // NS
    n_ob = e2 // OMCH
    assert n_ob % 2 == 0
    n_grp = OMCH // LANES
    rows_per_tile = rng // NS
    n_init = rows_per_tile // CH
    bin_cap = BIN + OMCH + CH
    n_pass = (n_ranges + NC - 1) // NC
    mesh = plsc.VectorSubcoreMesh(core_axis_name="c", subcore_axis_name="s")

    @functools.partial(
        pl.kernel,
        mesh=mesh,
        compiler_params=_SC_PARAMS,
        out_type=jax.ShapeDtypeStruct((n_pad, c), jnp.float32),
        scratch_types=[
            pltpu.VMEM((OMCH,), jnp.int32),
            pltpu.VMEM((OMCH,), jnp.int32),
            pltpu.VMEM((bin_cap,), jnp.int32),
            pltpu.VMEM((bin_cap,), jnp.int32),
            pltpu.VMEM((CH,), jnp.int32),
            pltpu.VMEM((CH,), jnp.int32),
            pltpu.VMEM((CH,), jnp.int32),
            pltpu.VMEM((CH, c), jnp.float32),
            pltpu.VMEM((CH, c), jnp.float32),
            pltpu.VMEM((CH, c), jnp.float32),
            pltpu.VMEM((c,), jnp.float32),
            pltpu.VMEM_SHARED((rng + 8, c), jnp.float32),
            pltpu.SemaphoreType.DMA,
            pltpu.SemaphoreType.DMA,
            pltpu.SemaphoreType.DMA,
            pltpu.SemaphoreType.DMA,
            pltpu.SemaphoreType.DMA,
            pltpu.SemaphoreType.DMA,
            pltpu.SemaphoreType.DMA,
            pltpu.SemaphoreType.DMA,
        ],
    )
    def scatter_k(contrib_hbm, omap_hbm, bias_hbm, out_hbm,
                  om_c0, om_c1, bin_idx, bin_off, off0, off1, off2,
                  rows0, rows1, rows2, bias_v, spmem,
                  osem0, osem1, gsem0, gsem1, gsem2,
                  asem0, asem1, asem2):
        cid = lax.axis_index("c")
        sid = lax.axis_index("s")
        share = sid * e2
        trash = jnp.int32(rng)
        iota = lax.iota(jnp.int32, LANES)
        dump = jnp.int32(bin_cap - LANES) + iota
        om_bufs = (om_c0, om_c1)
        om_sems = (osem0, osem1)
        rows = (rows0, rows1, rows2)
        offs = (off0, off1, off2)
        gsems = (gsem0, gsem1, gsem2)
        asems = (asem0, asem1, asem2)

        pltpu.sync_copy(bias_hbm, bias_v)

        def g_issue(jc, b):
            pltpu.async_copy(
                contrib_hbm.at[bin_idx.at[pl.ds(jc * CH, CH)]],
                rows[b], gsems[b],
            )

        def g_wait(b):
            pltpu.make_async_copy(
                contrib_hbm.at[pl.ds(0, CH)], rows[b], gsems[b]
            ).wait()

        def do_add(jc, b):
            for i in range(CH // LANES):
                offs[b][pl.ds(i * LANES, LANES)] = bin_off[
                    pl.ds(jc * CH + i * LANES, LANES)
                ]
            pltpu.async_copy(rows[b], spmem.at[offs[b]], asems[b], add=True)

        def a_wait(b):
            pltpu.make_async_copy(rows[b], spmem.at[pl.ds(0, CH)],
                                  asems[b]).wait()

        def emit(nch):
            @pl.when(nch > 0)
            def _():
                g_issue(0, 0)

            @pl.when(nch > 1)
            def _():
                g_issue(1, 1)

            def handle(j, b):
                @pl.when(j < nch)
                def _():
                    g_wait(b)
                    do_add(j, b)

                    @pl.when(j + 2 < nch)
                    def _():
                        b2 = (b + 2) % 3

                        @pl.when(j >= 1)
                        def _():
                            a_wait(b2)

                        g_issue(j + 2, b2)

            @pl.loop(0, (nch + 2) // 3)
            def _(t):
                for i in range(3):
                    handle(3 * t + i, i)

            for b in range(3):
                @pl.when(nch > b)
                def _():
                    a_wait(b)

        def pad_and_emit(cnt):
            for i in range(CH // LANES):
                pos = cnt + i * LANES + iota
                plsc.store_scatter(bin_idx, [pos],
                                   jnp.zeros((LANES,), jnp.int32))
                plsc.store_scatter(bin_off, [pos],
                                   jnp.full((LANES,), trash, jnp.int32))
            emit((cnt + CH - 1) // CH)

        def om_issue(ob, h):
            pltpu.async_copy(
                omap_hbm.at[pl.ds(share + ob * OMCH, OMCH)],
                om_bufs[h], om_sems[h],
            )

        def om_wait(h):
            pltpu.make_async_copy(
                omap_hbm.at[pl.ds(0, OMCH)], om_bufs[h], om_sems[h]
            ).wait()

        @pl.loop(0, n_pass)
        def _(p):
            r = p * NC + cid

            @pl.when(r < n_ranges)
            def _():
                base = r * rng

                @pl.loop(0, CH)
                def _(i):
                    for j in range(c // LANES):
                        rows0[i, pl.ds(j * LANES, LANES)] = bias_v[
                            pl.ds(j * LANES, LANES)
                        ]

                for ch in range(n_init):
                    pltpu.sync_copy(
                        rows0,
                        spmem.at[pl.ds(sid * rows_per_tile + ch * CH, CH)],
                    )
                plsc.subcore_barrier()

                om_issue(0, 0)

                @pl.loop(0, n_ob // 2,
                         init_carry=jnp.zeros((LANES,), jnp.int32))
                def scan(u, cnt_v):
                    for h in range(2):
                        ob = 2 * u + h

                        @pl.when(ob + 1 < n_ob)
                        def _():
                            om_issue(ob + 1, 1 - h)

                        om_wait(h)
                        om_c = om_bufs[h]

                        @pl.loop(0, n_grp, init_carry=cnt_v)
                        def grp(g, cnt_v):
                            v = om_c[pl.ds(g * LANES, LANES)]
                            local = v - base
                            mask = (local >= 0) & (local < rng)
                            gidx = share + ob * OMCH + g * LANES + iota
                            csum = plsc.cumsum(mask.astype(jnp.int32))
                            pos = jnp.where(mask, cnt_v + csum - 1, dump)
                            plsc.store_scatter(bin_idx, [pos], gidx)
                            plsc.store_scatter(
                                bin_off, [pos],
                                jnp.where(mask, local, trash),
                            )
                            return cnt_v + plsc.all_reduce_population_count(
                                mask
                            )

                        cnt_v = grp
                        s = jnp.max(cnt_v)

                        @pl.when(s >= BIN)
                        def _():
                            pad_and_emit(s)

                        cnt_v = jnp.where(
                            jnp.broadcast_to(s >= BIN, (LANES,)),
                            jnp.zeros((LANES,), jnp.int32),
                            cnt_v,
                        )
                    return cnt_v

                pad_and_emit(jnp.max(scan))
                plsc.subcore_barrier()

                row0 = sid * rows_per_tile
                pltpu.sync_copy(
                    spmem.at[pl.ds(row0, rows_per_tile)],
                    out_hbm.at[pl.ds(base + row0, rows_per_tile)],
                )

    return scatter_k


def kernel(features, in_map, out_map, weight, bias):
    n_feat, c = features.shape
    k, m = in_map.shape

    mp = m
    while (k * mp) % (128 * NW) or (k * mp // NS) % OMCH:
        mp += 1
    flat = k * mp
    e = flat // NW

    rng = 10240
    n_ranges = -(-n_feat // rng)
    n_pad = n_ranges * rng

    in_p = jnp.pad(in_map, ((0, 0), (0, mp - m))).reshape(-1)
    om_p = jnp.pad(
        out_map, ((0, 0), (0, mp - m)), constant_values=SENTINEL
    ).reshape(-1)

    gathered = _make_gather(c, e)(features, in_p)
    contrib = _gemm(gathered, weight, k, mp, c)
    out_pad = _make_scatter(c, flat, rng, n_ranges, n_pad)(
        contrib, om_p, bias
    )
    return out_pad[:n_feat]
